# baseline (speedup 1.0000x reference)
import functools

import jax
import jax.numpy as jnp
from jax.experimental import pallas as pl
from jax.experimental.pallas import tpu as pltpu

NUM_EXPERTS = 8
TOP_K = 2
D_MODEL = 1024
D_FF = 512
T = 2048

BT = 256


def _dense_body(x_ref, gate_ref, wgu_ref, wdn_ref, out_ref, acc_ref):
    e = pl.program_id(1)

    x = x_ref[...]

    logits = jnp.dot(x, gate_ref[...], preferred_element_type=jnp.float32)
    m = jnp.max(logits, axis=-1, keepdims=True)
    ex = jnp.exp(logits - m)
    probs = ex / jnp.sum(ex, axis=-1, keepdims=True)

    idx = jax.lax.broadcasted_iota(jnp.int32, probs.shape, 1)
    m1 = jnp.max(probs, axis=-1, keepdims=True)
    i1 = jnp.min(jnp.where(probs == m1, idx, NUM_EXPERTS), axis=-1, keepdims=True)
    mask1 = idx == i1
    probs2 = jnp.where(mask1, -jnp.inf, probs)
    m2 = jnp.max(probs2, axis=-1, keepdims=True)
    i2 = jnp.min(jnp.where(probs2 == m2, idx, NUM_EXPERTS), axis=-1, keepdims=True)
    mask2 = idx == i2
    combine = jnp.where(mask1 | mask2, probs, 0.0)

    xb = x.astype(jnp.bfloat16)
    gu = jnp.dot(xb, wgu_ref[0].astype(jnp.bfloat16),
                 preferred_element_type=jnp.float32)
    g = gu[:, :D_FF]
    u = gu[:, D_FF:]
    act = (g * jax.lax.logistic(g)) * u
    o = jnp.dot(act.astype(jnp.bfloat16), wdn_ref[0].astype(jnp.bfloat16),
                preferred_element_type=jnp.float32)

    w_e = jnp.sum(jnp.where(idx == e, combine, 0.0), axis=-1, keepdims=True)
    contrib = w_e * o

    @pl.when(e == 0)
    def _():
        acc_ref[...] = contrib

    @pl.when(e > 0)
    def _():
        acc_ref[...] += contrib

    @pl.when(e == NUM_EXPERTS - 1)
    def _():
        out_ref[...] = acc_ref[...]


def kernel(hidden_states, num_global_tokens, max_num_tokens_per_gpu, gate_w, w_gate_up, w_down):
    nT = T // BT
    out = pl.pallas_call(
        _dense_body,
        grid=(nT, NUM_EXPERTS),
        in_specs=[
            pl.BlockSpec((BT, D_MODEL), lambda i, e: (i, 0)),
            pl.BlockSpec((D_MODEL, NUM_EXPERTS), lambda i, e: (0, 0)),
            pl.BlockSpec((1, D_MODEL, 2 * D_FF), lambda i, e: (e, 0, 0)),
            pl.BlockSpec((1, D_FF, D_MODEL), lambda i, e: (e, 0, 0)),
        ],
        out_specs=pl.BlockSpec((BT, D_MODEL), lambda i, e: (i, 0)),
        out_shape=jax.ShapeDtypeStruct((T, D_MODEL), jnp.float32),
        scratch_shapes=[pltpu.VMEM((BT, D_MODEL), jnp.float32)],
        compiler_params=pltpu.CompilerParams(
            dimension_semantics=("parallel", "arbitrary"),
        ),
    )(hidden_states, gate_w, w_gate_up, w_down)
    return out


# speedup vs baseline: 1.3418x; 1.3418x over previous
import jax
import jax.numpy as jnp
from jax.experimental import pallas as pl
from jax.experimental.pallas import tpu as pltpu

NUM_EXPERTS = 8
TOP_K = 2
D_MODEL = 1024
D_FF = 512
T = 2048


def _dense_body(x_ref, gate_ref, wgu_ref, wdn_ref, out_ref, comb_ref):
    e = pl.program_id(0)

    x = x_ref[...]

    @pl.when(e == 0)
    def _():
        logits = jnp.dot(x, gate_ref[...], preferred_element_type=jnp.float32,
                         precision=jax.lax.Precision.HIGHEST)
        m = jnp.max(logits, axis=-1, keepdims=True)
        ex = jnp.exp(logits - m)
        probs = ex / jnp.sum(ex, axis=-1, keepdims=True)

        idx = jax.lax.broadcasted_iota(jnp.int32, probs.shape, 1)
        m1 = jnp.max(probs, axis=-1, keepdims=True)
        i1 = jnp.min(jnp.where(probs == m1, idx, NUM_EXPERTS), axis=-1, keepdims=True)
        mask1 = idx == i1
        probs2 = jnp.where(mask1, -jnp.inf, probs)
        m2 = jnp.max(probs2, axis=-1, keepdims=True)
        i2 = jnp.min(jnp.where(probs2 == m2, idx, NUM_EXPERTS), axis=-1, keepdims=True)
        mask2 = idx == i2
        comb_ref[...] = jnp.where(mask1 | mask2, probs, 0.0)

    xb = x.astype(jnp.bfloat16)
    gu = jnp.dot(xb, wgu_ref[0], preferred_element_type=jnp.float32)
    g = gu[:, :D_FF]
    u = gu[:, D_FF:]
    act = (g * jax.lax.logistic(g)) * u
    o = jnp.dot(act.astype(jnp.bfloat16), wdn_ref[0],
                preferred_element_type=jnp.float32)

    idx2 = jax.lax.broadcasted_iota(jnp.int32, (T, NUM_EXPERTS), 1)
    w_e = jnp.sum(jnp.where(idx2 == e, comb_ref[...], 0.0), axis=-1, keepdims=True)
    contrib = w_e * o

    @pl.when(e == 0)
    def _():
        out_ref[...] = contrib

    @pl.when(e > 0)
    def _():
        out_ref[...] += contrib


def kernel(hidden_states, num_global_tokens, max_num_tokens_per_gpu, gate_w, w_gate_up, w_down):
    wgu = w_gate_up.astype(jnp.bfloat16)
    wdn = w_down.astype(jnp.bfloat16)
    out = pl.pallas_call(
        _dense_body,
        grid=(NUM_EXPERTS,),
        in_specs=[
            pl.BlockSpec((T, D_MODEL), lambda e: (0, 0)),
            pl.BlockSpec((D_MODEL, NUM_EXPERTS), lambda e: (0, 0)),
            pl.BlockSpec((1, D_MODEL, 2 * D_FF), lambda e: (e, 0, 0)),
            pl.BlockSpec((1, D_FF, D_MODEL), lambda e: (e, 0, 0)),
        ],
        out_specs=pl.BlockSpec((T, D_MODEL), lambda e: (0, 0)),
        out_shape=jax.ShapeDtypeStruct((T, D_MODEL), jnp.float32),
        scratch_shapes=[pltpu.VMEM((T, NUM_EXPERTS), jnp.float32)],
        compiler_params=pltpu.CompilerParams(
            dimension_semantics=("arbitrary",),
        ),
    )(hidden_states, gate_w, wgu, wdn)
    return out


# speedup vs baseline: 1.4114x; 1.0519x over previous
import jax
import jax.numpy as jnp
from jax.experimental import pallas as pl
from jax.experimental.pallas import tpu as pltpu

NUM_EXPERTS = 8
TOP_K = 2
D_MODEL = 1024
D_FF = 512
T = 2048


def _dense_body(x_ref, gate_ref, wgu_ref, wdn_ref, out_ref, comb_ref):
    e = pl.program_id(0)

    x = x_ref[...]

    @pl.when(e == 0)
    def _():
        logits = jnp.dot(x, gate_ref[...], preferred_element_type=jnp.float32)
        m = jnp.max(logits, axis=-1, keepdims=True)
        ex = jnp.exp(logits - m)
        probs = ex / jnp.sum(ex, axis=-1, keepdims=True)

        idx = jax.lax.broadcasted_iota(jnp.int32, probs.shape, 1)
        m1 = jnp.max(probs, axis=-1, keepdims=True)
        i1 = jnp.min(jnp.where(probs == m1, idx, NUM_EXPERTS), axis=-1, keepdims=True)
        mask1 = idx == i1
        probs2 = jnp.where(mask1, -jnp.inf, probs)
        m2 = jnp.max(probs2, axis=-1, keepdims=True)
        i2 = jnp.min(jnp.where(probs2 == m2, idx, NUM_EXPERTS), axis=-1, keepdims=True)
        mask2 = idx == i2
        comb_ref[...] = jnp.where(mask1 | mask2, probs, 0.0)

    xb = x.astype(jnp.bfloat16)
    gu = jnp.dot(xb, wgu_ref[0], preferred_element_type=jnp.float32)
    g = gu[:, :D_FF]
    u = gu[:, D_FF:]
    act = (g * jax.lax.logistic(g)) * u
    o = jnp.dot(act.astype(jnp.bfloat16), wdn_ref[0],
                preferred_element_type=jnp.float32)

    idx2 = jax.lax.broadcasted_iota(jnp.int32, (T, NUM_EXPERTS), 1)
    w_e = jnp.sum(jnp.where(idx2 == e, comb_ref[...], 0.0), axis=-1, keepdims=True)
    contrib = w_e * o

    @pl.when(e == 0)
    def _():
        out_ref[...] = contrib

    @pl.when(e > 0)
    def _():
        out_ref[...] += contrib


def kernel(hidden_states, num_global_tokens, max_num_tokens_per_gpu, gate_w, w_gate_up, w_down):
    wgu = w_gate_up.astype(jnp.bfloat16)
    wdn = w_down.astype(jnp.bfloat16)
    out = pl.pallas_call(
        _dense_body,
        grid=(NUM_EXPERTS,),
        in_specs=[
            pl.BlockSpec((T, D_MODEL), lambda e: (0, 0)),
            pl.BlockSpec((D_MODEL, NUM_EXPERTS), lambda e: (0, 0)),
            pl.BlockSpec((1, D_MODEL, 2 * D_FF), lambda e: (e, 0, 0)),
            pl.BlockSpec((1, D_FF, D_MODEL), lambda e: (e, 0, 0)),
        ],
        out_specs=pl.BlockSpec((T, D_MODEL), lambda e: (0, 0)),
        out_shape=jax.ShapeDtypeStruct((T, D_MODEL), jnp.float32),
        scratch_shapes=[pltpu.VMEM((T, NUM_EXPERTS), jnp.float32)],
        compiler_params=pltpu.CompilerParams(
            dimension_semantics=("arbitrary",),
        ),
    )(hidden_states, gate_w, wgu, wdn)
    return out
